# baseline (speedup 1.0000x reference)
import functools

import jax
import jax.numpy as jnp
import numpy as np
from jax import lax
from jax.experimental import pallas as pl
from jax.experimental.pallas import tpu as pltpu
from jax.experimental.pallas import tpu_sc as plsc

N = 10000
E = 320000
D = 128
ED = 7

NC = 2
NS = 16
L = 16
NW = NC * NS

EB = 80
EPT = E // NW
NB = EPT // EB
NPAD = 10240
ZROWS = NPAD // NS
OROWS = 624
NSEG = D // L


def _worker_id():
  return lax.axis_index("s") * NC + lax.axis_index("c")


def _sc_agg_body(h_hbm, ep_hbm, src_hbm, dst_hbm, out_hbm,
                 src0, src1, dst0, dst1, dsc0, dsc1, ep0, ep1, rows0, rows1,
                 acc_sh, si0, si1, sg0, sg1, so0, so1):
  c = lax.axis_index("c")
  s = lax.axis_index("s")
  wid = _worker_id()
  srcs, dsts, eps, rows = (src0, src1), (dst0, dst1), (ep0, ep1), (rows0, rows1)
  dscs = (dsc0, dsc1)
  sem_idx, sem_in, sem_out = (si0, si1), (sg0, sg1), (so0, so1)

  def off(g):
    return wid * EPT + g * EB

  def issue_idx(g, p):
    pltpu.async_copy(src_hbm.at[pl.ds(off(g), EB)], srcs[p], sem_idx[p])
    pltpu.async_copy(dst_hbm.at[pl.ds(off(g), EB)], dsts[p], sem_idx[p])

  def drain_idx(p):
    pltpu.make_async_copy(src_hbm.at[pl.ds(0, EB)], srcs[p], sem_idx[p]).wait()
    pltpu.make_async_copy(dst_hbm.at[pl.ds(0, EB)], dsts[p], sem_idx[p]).wait()

  def issue_in(g, p):
    pltpu.async_copy(h_hbm.at[srcs[p]], rows[p], sem_in[p])
    pltpu.async_copy(ep_hbm.at[pl.ds(off(g), EB)], eps[p], sem_in[p])

  def drain_in(p):
    pltpu.make_async_copy(h_hbm.at[srcs[p]], rows[p], sem_in[p]).wait()
    pltpu.make_async_copy(ep_hbm.at[pl.ds(0, EB)], eps[p], sem_in[p]).wait()

  def drain_out(p):
    pltpu.make_async_copy(rows[p], acc_sh.at[dscs[p]], sem_out[p]).wait()

  def copy_dst(p):
    for k in range(EB // L):
      dscs[p][pl.ds(k * L, L)] = dsts[p][pl.ds(k * L, L)]

  def compute(p):
    def edge(i, _):
      for k2 in range(NSEG // 2):
        ab = eps[p][i, pl.ds(k2 * 2 * L, 2 * L)]
        e0, e1 = plsc.unpack(ab, format=plsc.PackFormat.INTERLEAVED)
        s0 = pl.ds(2 * k2 * L, L)
        s1 = pl.ds((2 * k2 + 1) * L, L)
        rows[p][i, s0] = jnp.maximum(rows[p][i, s0] + e0, 0.0)
        rows[p][i, s1] = jnp.maximum(rows[p][i, s1] + e1, 0.0)
      return 0
    lax.fori_loop(0, EB, edge, 0, unroll=False)

  def zrow(i, _):
    for k in range(NSEG):
      rows0[i, pl.ds(k * L, L)] = jnp.zeros((L,), jnp.float32)
    return 0
  lax.fori_loop(0, EB, zrow, 0, unroll=False)

  def zchunk(j, _):
    pltpu.sync_copy(rows0, acc_sh.at[pl.ds(s * ZROWS + j * EB, EB)])
    return 0
  lax.fori_loop(0, ZROWS // EB, zchunk, 0, unroll=False)
  plsc.subcore_barrier()

  issue_idx(0, 0)
  issue_idx(1, 1)
  drain_idx(0)
  issue_in(0, 0)

  def pair(gi, _):
    for p in range(2):
      g = gi * 2 + p
      q = 1 - p
      drain_in(p)
      copy_dst(p)

      @pl.when(g > 0)
      def _():
        drain_out(q)

      @pl.when(g < NB - 1)
      def _():
        drain_idx(q)
        issue_in(g + 1, q)

      @pl.when(g < NB - 2)
      def _():
        issue_idx(g + 2, p)

      compute(p)
      pltpu.async_copy(rows[p], acc_sh.at[dscs[p]], sem_out[p], add=True)
    return 0

  lax.fori_loop(0, NB // 2, pair, 0, unroll=False)

  drain_in(0)
  copy_dst(0)
  drain_out(1)
  compute(0)
  pltpu.sync_copy(rows0, acc_sh.at[dsc0], add=True)

  plsc.subcore_barrier()
  pltpu.sync_copy(acc_sh.at[pl.ds(s * OROWS, OROWS)],
                  out_hbm.at[c, pl.ds(s * OROWS, OROWS)])

  @pl.when(s == NS - 1)
  def _():
    pltpu.sync_copy(acc_sh.at[pl.ds(NS * OROWS, N - NS * OROWS)],
                    out_hbm.at[c, pl.ds(NS * OROWS, N - NS * OROWS)])


@jax.jit
def _sc_agg(h, ep, src, dst):
  return pl.kernel(
      _sc_agg_body,
      out_type=jax.ShapeDtypeStruct((NC, N, D), jnp.float32),
      mesh=plsc.VectorSubcoreMesh(core_axis_name="c", subcore_axis_name="s", num_cores=NC, num_subcores=NS),
      compiler_params=pltpu.CompilerParams(needs_layout_passes=False),
      scratch_types=(
          [pltpu.VMEM((EB,), jnp.int32)] * 6
          + [pltpu.VMEM((EB, D), jnp.bfloat16)] * 2
          + [pltpu.VMEM((EB, D), jnp.float32)] * 2
          + [pltpu.VMEM_SHARED((NPAD, D), jnp.float32)]
          + [pltpu.SemaphoreType.DMA] * 6
      ),
  )(h, ep, src, dst)


def _sc_edge_body(p_hbm, q_hbm, r_hbm, src_hbm, dst_hbm, w2_hbm, b2_hbm,
                  o0_hbm, o1_hbm,
                  src0, src1, dst0, dst1, pv0, pv1, qv0, qv1, rv0, rv1,
                  w2_v, b2_v, m0_v, m1_v, o00, o01, o10, o11,
                  si0, si1, sg0, sg1, so0, so1):
  wid = _worker_id()
  srcs, dsts = (src0, src1), (dst0, dst1)
  pvs, qvs, rvs = (pv0, pv1), (qv0, qv1), (rv0, rv1)
  o0s, o1s = (o00, o01), (o10, o11)
  sem_idx, sem_in, sem_out = (si0, si1), (sg0, sg1), (so0, so1)
  pltpu.sync_copy(w2_hbm, w2_v)
  pltpu.sync_copy(b2_hbm, b2_v)

  def off(g):
    return wid * EPT + g * EB

  def issue_idx(g, p):
    pltpu.async_copy(src_hbm.at[pl.ds(off(g), EB)], srcs[p], sem_idx[p])
    pltpu.async_copy(dst_hbm.at[pl.ds(off(g), EB)], dsts[p], sem_idx[p])

  def drain_idx(p):
    pltpu.make_async_copy(src_hbm.at[pl.ds(0, EB)], srcs[p], sem_idx[p]).wait()
    pltpu.make_async_copy(dst_hbm.at[pl.ds(0, EB)], dsts[p], sem_idx[p]).wait()

  def issue_in(g, p):
    pltpu.async_copy(p_hbm.at[srcs[p]], pvs[p], sem_in[p])
    pltpu.async_copy(q_hbm.at[dsts[p]], qvs[p], sem_in[p])
    pltpu.async_copy(r_hbm.at[pl.ds(off(g), EB)], rvs[p], sem_in[p])

  def drain_in(p):
    pltpu.make_async_copy(p_hbm.at[srcs[p]], pvs[p], sem_in[p]).wait()
    pltpu.make_async_copy(q_hbm.at[dsts[p]], qvs[p], sem_in[p]).wait()
    pltpu.make_async_copy(r_hbm.at[pl.ds(0, EB)], rvs[p], sem_in[p]).wait()

  def drain_out(p):
    pltpu.make_async_copy(o0s[p], o0_hbm.at[pl.ds(0, EB)], sem_out[p]).wait()
    pltpu.make_async_copy(o1s[p], o1_hbm.at[pl.ds(0, EB)], sem_out[p]).wait()

  def compute(p):
    def edge(i, _):
      acc0 = jnp.zeros((L,), jnp.float32)
      acc1 = jnp.zeros((L,), jnp.float32)
      for k2 in range(NSEG // 2):
        sl2 = pl.ds(k2 * 2 * L, 2 * L)
        r0, r1 = plsc.unpack(rvs[p][i, sl2], format=plsc.PackFormat.INTERLEAVED)
        s0 = pl.ds(2 * k2 * L, L)
        s1 = pl.ds((2 * k2 + 1) * L, L)
        z0 = jnp.maximum(pvs[p][i, s0] + qvs[p][i, s0] + r0, 0.0)
        z1 = jnp.maximum(pvs[p][i, s1] + qvs[p][i, s1] + r1, 0.0)
        acc0 = acc0 + z0 * w2_v[0, s0] + z1 * w2_v[0, s1]
        acc1 = acc1 + z0 * w2_v[1, s0] + z1 * w2_v[1, s1]
      m0_v[pl.ds(i * L, L)] = acc0
      m1_v[pl.ds(i * L, L)] = acc1
      return 0
    lax.fori_loop(0, EB, edge, 0, unroll=False)

    def grp(jg, _):
      rows = (jnp.arange(L, dtype=jnp.int32) + jg * L) * L
      v0 = b2_v[0, :]
      v1 = b2_v[1, :]
      for k in range(L):
        idx = rows + k
        v0 = v0 + plsc.load_gather(m0_v, [idx])
        v1 = v1 + plsc.load_gather(m1_v, [idx])
      o0s[p][pl.ds(jg * L, L)] = v0
      o1s[p][pl.ds(jg * L, L)] = v1
      return 0
    lax.fori_loop(0, EB // L, grp, 0, unroll=False)

  issue_idx(0, 0)
  issue_idx(1, 1)
  drain_idx(0)
  issue_in(0, 0)

  def pair(gi, _):
    for p in range(2):
      g = gi * 2 + p
      q = 1 - p
      drain_in(p)

      @pl.when(g < NB - 1)
      def _():
        drain_idx(q)
        issue_in(g + 1, q)

      @pl.when(g < NB - 2)
      def _():
        issue_idx(g + 2, p)

      @pl.when(g > 1)
      def _():
        drain_out(p)

      compute(p)
      pltpu.async_copy(o0s[p], o0_hbm.at[pl.ds(off(g), EB)], sem_out[p])
      pltpu.async_copy(o1s[p], o1_hbm.at[pl.ds(off(g), EB)], sem_out[p])
    return 0

  lax.fori_loop(0, NB // 2, pair, 0, unroll=False)

  g = NB - 1
  drain_in(0)
  drain_out(0)
  compute(0)
  pltpu.sync_copy(o00, o0_hbm.at[pl.ds(off(g), EB)])
  pltpu.sync_copy(o10, o1_hbm.at[pl.ds(off(g), EB)])
  drain_out(1)


@jax.jit
def _sc_edge(p, q, r, src, dst, w2, b2v):
  return pl.kernel(
      _sc_edge_body,
      out_type=[jax.ShapeDtypeStruct((E,), jnp.float32),
                jax.ShapeDtypeStruct((E,), jnp.float32)],
      mesh=plsc.VectorSubcoreMesh(core_axis_name="c", subcore_axis_name="s", num_cores=NC, num_subcores=NS),
      compiler_params=pltpu.CompilerParams(needs_layout_passes=False),
      scratch_types=(
          [pltpu.VMEM((EB,), jnp.int32)] * 4
          + [pltpu.VMEM((EB, D), jnp.float32)] * 4
          + [pltpu.VMEM((EB, D), jnp.bfloat16)] * 2
          + [pltpu.VMEM((2, D), jnp.float32)]
          + [pltpu.VMEM((2, L), jnp.float32)]
          + [pltpu.VMEM((EB * L,), jnp.float32)] * 2
          + [pltpu.VMEM((EB,), jnp.float32)] * 4
          + [pltpu.SemaphoreType.DMA] * 6
      ),
  )(p, q, r, src, dst, w2, b2v)


_EBLK = 2000


def _tc_edge_lin1_body(ea_ref, w_ref, b_ref, o1_ref):
  o1_ref[...] = (jnp.dot(ea_ref[...], w_ref[...],
                         preferred_element_type=jnp.float32)
                 + b_ref[...]).astype(jnp.bfloat16)


def _tc_edge_lin2_body(ea_ref, w_ref, b_ref, o2_ref, o3_ref):
  acc = (jnp.dot(ea_ref[...], w_ref[...],
                 preferred_element_type=jnp.float32)
         + b_ref[...]).astype(jnp.bfloat16)
  o2_ref[...] = acc[:, :D]
  o3_ref[...] = acc[:, D:]


_e_out = jax.ShapeDtypeStruct((E, D), jnp.bfloat16)


@jax.jit
def _tc_edge_lin1(ea_pad, w, b):
  return pl.pallas_call(
      _tc_edge_lin1_body,
      grid=(E // _EBLK,),
      in_specs=[
          pl.BlockSpec((_EBLK, 8), lambda i: (i, 0)),
          pl.BlockSpec((8, D), lambda i: (0, 0)),
          pl.BlockSpec((1, D), lambda i: (0, 0)),
      ],
      out_specs=pl.BlockSpec((_EBLK, D), lambda i: (i, 0)),
      out_shape=_e_out,
  )(ea_pad, w, b)


@jax.jit
def _tc_edge_lin2(ea_pad, w, b):
  return pl.pallas_call(
      _tc_edge_lin2_body,
      grid=(E // _EBLK,),
      in_specs=[
          pl.BlockSpec((_EBLK, 8), lambda i: (i, 0)),
          pl.BlockSpec((8, 2 * D), lambda i: (0, 0)),
          pl.BlockSpec((1, 2 * D), lambda i: (0, 0)),
      ],
      out_specs=[
          pl.BlockSpec((_EBLK, D), lambda i: (i, 0)),
          pl.BlockSpec((_EBLK, D), lambda i: (i, 0)),
      ],
      out_shape=[_e_out, _e_out],
  )(ea_pad, w, b)


_NBLK_TC = 1000


def _tc_mlp_body(x_ref, p_ref, w1_ref, b1_ref, w2_ref, b2_ref, o_ref):
  h = x_ref[...] + p_ref[0] + p_ref[1]
  t = jax.nn.relu(jnp.dot(h, w1_ref[...],
                          preferred_element_type=jnp.float32) + b1_ref[...])
  o_ref[...] = jax.nn.relu(jnp.dot(t, w2_ref[...],
                                   preferred_element_type=jnp.float32)
                           + b2_ref[...])


def _tc_mlp2_body(x_ref, p_ref, w1_ref, b1_ref, w2_ref, b2_ref,
                  ws_ref, wd_ref, o_ref, po_ref, qo_ref):
  h = x_ref[...] + p_ref[0] + p_ref[1]
  t = jax.nn.relu(jnp.dot(h, w1_ref[...],
                          preferred_element_type=jnp.float32) + b1_ref[...])
  o = jax.nn.relu(jnp.dot(t, w2_ref[...],
                          preferred_element_type=jnp.float32) + b2_ref[...])
  o_ref[...] = o
  po_ref[...] = jnp.dot(o, ws_ref[...], preferred_element_type=jnp.float32)
  qo_ref[...] = jnp.dot(o, wd_ref[...], preferred_element_type=jnp.float32)


_mat_spec = pl.BlockSpec((D, D), lambda i: (0, 0))
_bias_spec = pl.BlockSpec((1, D), lambda i: (0, 0))
_row_spec = pl.BlockSpec((_NBLK_TC, D), lambda i: (i, 0))
_part_spec = pl.BlockSpec((NC, _NBLK_TC, D), lambda i: (0, i, 0))
_n_out = jax.ShapeDtypeStruct((N, D), jnp.float32)


@jax.jit
def _tc_mlp(x, part, w1t, b1, w2t, b2):
  return pl.pallas_call(
      _tc_mlp_body,
      grid=(N // _NBLK_TC,),
      in_specs=[_row_spec, _part_spec, _mat_spec, _bias_spec, _mat_spec,
                _bias_spec],
      out_specs=_row_spec,
      out_shape=_n_out,
  )(x, part, w1t, b1, w2t, b2)


@jax.jit
def _tc_mlp2(x, part, w1t, b1, w2t, b2, wst, wdt):
  return pl.pallas_call(
      _tc_mlp2_body,
      grid=(N // _NBLK_TC,),
      in_specs=[_row_spec, _part_spec, _mat_spec, _bias_spec, _mat_spec,
                _bias_spec, _mat_spec, _mat_spec],
      out_specs=[_row_spec, _row_spec, _row_spec],
      out_shape=[_n_out, _n_out, _n_out],
  )(x, part, w1t, b1, w2t, b2, wst, wdt)


_SRCIDX = np.concatenate([
    np.stack([np.arange(16) + 32 * k2, np.arange(16) + 32 * k2 + 16],
             axis=1).reshape(-1)
    for k2 in range(D // 32)
])


def kernel(x, edge_index, edge_attr, e1_W, e1_b, n1_W1, n1_b1, n1_W2, n1_b2,
           e2_W, e2_b, n2_W1, n2_b1, n2_W2, n2_b2, m_W1, m_b1, m_W2, m_b2):
  src = edge_index[0]
  dst = edge_index[1]

  ea_pad = jnp.pad(edge_attr, ((0, 0), (0, 8 - ED)))
  we_t = m_W1[:, 2 * D:].T
  w1p = jnp.pad(e1_W.T[:, _SRCIDX], ((0, 1), (0, 0)))
  w23 = jnp.pad(
      jnp.concatenate([e2_W.T[:, _SRCIDX], we_t[:, _SRCIDX]], axis=1),
      ((0, 1), (0, 0)))
  b23 = jnp.concatenate([e2_b[_SRCIDX], m_b1[_SRCIDX]])[None, :]

  ep1 = _tc_edge_lin1(ea_pad, w1p, e1_b[_SRCIDX][None, :])

  part1 = _sc_agg(x, ep1, src, dst)
  ep2, r = _tc_edge_lin2(ea_pad, w23, b23)
  h1 = _tc_mlp(x, part1, n1_W1.T, n1_b1[None, :], n1_W2.T, n1_b2[None, :])

  part2 = _sc_agg(h1, ep2, src, dst)
  h2, p, q = _tc_mlp2(h1, part2, n2_W1.T, n2_b1[None, :], n2_W2.T,
                      n2_b2[None, :], m_W1[:, :D].T, m_W1[:, D:2 * D].T)

  b2v = jnp.broadcast_to(m_b2[:, None], (2, L))
  o0, o1 = _sc_edge(p, q, r, src, dst, m_W2, b2v)
  return jnp.stack([o0, o1], axis=1)


# speedup vs baseline: 1.0741x; 1.0741x over previous
import functools

import jax
import jax.numpy as jnp
import numpy as np
from jax import lax
from jax.experimental import pallas as pl
from jax.experimental.pallas import tpu as pltpu
from jax.experimental.pallas import tpu_sc as plsc

N = 10000
E = 320000
D = 128
ED = 7

NC = 2
NS = 16
L = 16
NW = NC * NS

EB = 80
EPT = E // NW
NB = EPT // EB
NPAD = 10240
ZROWS = NPAD // NS
OROWS = 624
NSEG = D // L


def _worker_id():
  return lax.axis_index("s") * NC + lax.axis_index("c")


def _sc_agg_body(h_hbm, ep_hbm, src_hbm, dst_hbm, out_hbm,
                 src0, src1, dst0, dst1, dsc0, dsc1, ep0, ep1, rows0, rows1,
                 acc_sh, si0, si1, sg0, sg1, so0, so1):
  c = lax.axis_index("c")
  s = lax.axis_index("s")
  wid = _worker_id()
  srcs, dsts, eps, rows = (src0, src1), (dst0, dst1), (ep0, ep1), (rows0, rows1)
  dscs = (dsc0, dsc1)
  sem_idx, sem_in, sem_out = (si0, si1), (sg0, sg1), (so0, so1)

  def off(g):
    return wid * EPT + g * EB

  def issue_idx(g, p):
    pltpu.async_copy(src_hbm.at[pl.ds(off(g), EB)], srcs[p], sem_idx[p])
    pltpu.async_copy(dst_hbm.at[pl.ds(off(g), EB)], dsts[p], sem_idx[p])

  def drain_idx(p):
    pltpu.make_async_copy(src_hbm.at[pl.ds(0, EB)], srcs[p], sem_idx[p]).wait()
    pltpu.make_async_copy(dst_hbm.at[pl.ds(0, EB)], dsts[p], sem_idx[p]).wait()

  def issue_in(g, p):
    pltpu.async_copy(h_hbm.at[srcs[p]], rows[p], sem_in[p])
    pltpu.async_copy(ep_hbm.at[pl.ds(off(g), EB)], eps[p], sem_in[p])

  def drain_in(p):
    pltpu.make_async_copy(h_hbm.at[srcs[p]], rows[p], sem_in[p]).wait()
    pltpu.make_async_copy(ep_hbm.at[pl.ds(0, EB)], eps[p], sem_in[p]).wait()

  def drain_out(p):
    pltpu.make_async_copy(rows[p], acc_sh.at[dscs[p]], sem_out[p]).wait()

  def copy_dst(p):
    for k in range(EB // L):
      dscs[p][pl.ds(k * L, L)] = dsts[p][pl.ds(k * L, L)]

  def compute(p):
    def edge(i, _):
      for k in range(NSEG):
        sl = pl.ds(k * L, L)
        rows[p][i, sl] = jnp.maximum(rows[p][i, sl] + eps[p][i, sl], 0.0)
      return 0
    lax.fori_loop(0, EB, edge, 0, unroll=False)

  def zrow(i, _):
    for k in range(NSEG):
      rows0[i, pl.ds(k * L, L)] = jnp.zeros((L,), jnp.float32)
    return 0
  lax.fori_loop(0, EB, zrow, 0, unroll=False)

  def zchunk(j, _):
    pltpu.sync_copy(rows0, acc_sh.at[pl.ds(s * ZROWS + j * EB, EB)])
    return 0
  lax.fori_loop(0, ZROWS // EB, zchunk, 0, unroll=False)
  plsc.subcore_barrier()

  issue_idx(0, 0)
  issue_idx(1, 1)
  drain_idx(0)
  issue_in(0, 0)

  def pair(gi, _):
    for p in range(2):
      g = gi * 2 + p
      q = 1 - p
      drain_in(p)
      copy_dst(p)

      @pl.when(g > 0)
      def _():
        drain_out(q)

      @pl.when(g < NB - 1)
      def _():
        drain_idx(q)
        issue_in(g + 1, q)

      @pl.when(g < NB - 2)
      def _():
        issue_idx(g + 2, p)

      compute(p)
      pltpu.async_copy(rows[p], acc_sh.at[dscs[p]], sem_out[p], add=True)
    return 0

  lax.fori_loop(0, NB // 2, pair, 0, unroll=False)

  drain_in(0)
  copy_dst(0)
  drain_out(1)
  compute(0)
  pltpu.sync_copy(rows0, acc_sh.at[dsc0], add=True)

  plsc.subcore_barrier()
  pltpu.sync_copy(acc_sh.at[pl.ds(s * OROWS, OROWS)],
                  out_hbm.at[c, pl.ds(s * OROWS, OROWS)])

  @pl.when(s == NS - 1)
  def _():
    pltpu.sync_copy(acc_sh.at[pl.ds(NS * OROWS, N - NS * OROWS)],
                    out_hbm.at[c, pl.ds(NS * OROWS, N - NS * OROWS)])


@jax.jit
def _sc_agg(h, ep, src, dst):
  return pl.kernel(
      _sc_agg_body,
      out_type=jax.ShapeDtypeStruct((NC, N, D), jnp.float32),
      mesh=plsc.VectorSubcoreMesh(core_axis_name="c", subcore_axis_name="s", num_cores=NC, num_subcores=NS),
      compiler_params=pltpu.CompilerParams(needs_layout_passes=False),
      scratch_types=(
          [pltpu.VMEM((EB,), jnp.int32)] * 6
          + [pltpu.VMEM((EB, D), jnp.float32)] * 4
          + [pltpu.VMEM_SHARED((NPAD, D), jnp.float32)]
          + [pltpu.SemaphoreType.DMA] * 6
      ),
  )(h, ep, src, dst)


def _sc_edge_body(p_hbm, q_hbm, r_hbm, src_hbm, dst_hbm, w2_hbm, b2_hbm,
                  o0_hbm, o1_hbm,
                  src0, src1, dst0, dst1, pv0, pv1, qv0, qv1, rv0, rv1,
                  w2_v, b2_v, m0_v, m1_v, o00, o01, o10, o11,
                  si0, si1, sg0, sg1, so0, so1):
  wid = _worker_id()
  srcs, dsts = (src0, src1), (dst0, dst1)
  pvs, qvs, rvs = (pv0, pv1), (qv0, qv1), (rv0, rv1)
  o0s, o1s = (o00, o01), (o10, o11)
  sem_idx, sem_in, sem_out = (si0, si1), (sg0, sg1), (so0, so1)
  pltpu.sync_copy(w2_hbm, w2_v)
  pltpu.sync_copy(b2_hbm, b2_v)

  def off(g):
    return wid * EPT + g * EB

  def issue_idx(g, p):
    pltpu.async_copy(src_hbm.at[pl.ds(off(g), EB)], srcs[p], sem_idx[p])
    pltpu.async_copy(dst_hbm.at[pl.ds(off(g), EB)], dsts[p], sem_idx[p])

  def drain_idx(p):
    pltpu.make_async_copy(src_hbm.at[pl.ds(0, EB)], srcs[p], sem_idx[p]).wait()
    pltpu.make_async_copy(dst_hbm.at[pl.ds(0, EB)], dsts[p], sem_idx[p]).wait()

  def issue_in(g, p):
    pltpu.async_copy(p_hbm.at[srcs[p]], pvs[p], sem_in[p])
    pltpu.async_copy(q_hbm.at[dsts[p]], qvs[p], sem_in[p])
    pltpu.async_copy(r_hbm.at[pl.ds(off(g), EB)], rvs[p], sem_in[p])

  def drain_in(p):
    pltpu.make_async_copy(p_hbm.at[srcs[p]], pvs[p], sem_in[p]).wait()
    pltpu.make_async_copy(q_hbm.at[dsts[p]], qvs[p], sem_in[p]).wait()
    pltpu.make_async_copy(r_hbm.at[pl.ds(0, EB)], rvs[p], sem_in[p]).wait()

  def drain_out(p):
    pltpu.make_async_copy(o0s[p], o0_hbm.at[pl.ds(0, EB)], sem_out[p]).wait()
    pltpu.make_async_copy(o1s[p], o1_hbm.at[pl.ds(0, EB)], sem_out[p]).wait()

  def compute(p):
    def edge(i, _):
      acc0 = jnp.zeros((L,), jnp.float32)
      acc1 = jnp.zeros((L,), jnp.float32)
      for k in range(NSEG):
        sl = pl.ds(k * L, L)
        z = jnp.maximum(pvs[p][i, sl] + qvs[p][i, sl] + rvs[p][i, sl], 0.0)
        acc0 = acc0 + z * w2_v[0, sl]
        acc1 = acc1 + z * w2_v[1, sl]
      m0_v[pl.ds(i * L, L)] = acc0
      m1_v[pl.ds(i * L, L)] = acc1
      return 0
    lax.fori_loop(0, EB, edge, 0, unroll=False)

    def grp(jg, _):
      rows = (jnp.arange(L, dtype=jnp.int32) + jg * L) * L
      v0 = b2_v[0, :]
      v1 = b2_v[1, :]
      for k in range(L):
        idx = rows + k
        v0 = v0 + plsc.load_gather(m0_v, [idx])
        v1 = v1 + plsc.load_gather(m1_v, [idx])
      o0s[p][pl.ds(jg * L, L)] = v0
      o1s[p][pl.ds(jg * L, L)] = v1
      return 0
    lax.fori_loop(0, EB // L, grp, 0, unroll=False)

  issue_idx(0, 0)
  issue_idx(1, 1)
  drain_idx(0)
  issue_in(0, 0)

  def pair(gi, _):
    for p in range(2):
      g = gi * 2 + p
      q = 1 - p
      drain_in(p)

      @pl.when(g < NB - 1)
      def _():
        drain_idx(q)
        issue_in(g + 1, q)

      @pl.when(g < NB - 2)
      def _():
        issue_idx(g + 2, p)

      @pl.when(g > 1)
      def _():
        drain_out(p)

      compute(p)
      pltpu.async_copy(o0s[p], o0_hbm.at[pl.ds(off(g), EB)], sem_out[p])
      pltpu.async_copy(o1s[p], o1_hbm.at[pl.ds(off(g), EB)], sem_out[p])
    return 0

  lax.fori_loop(0, NB // 2, pair, 0, unroll=False)

  g = NB - 1
  drain_in(0)
  drain_out(0)
  compute(0)
  pltpu.sync_copy(o00, o0_hbm.at[pl.ds(off(g), EB)])
  pltpu.sync_copy(o10, o1_hbm.at[pl.ds(off(g), EB)])
  drain_out(1)


@jax.jit
def _sc_edge(p, q, r, src, dst, w2, b2v):
  return pl.kernel(
      _sc_edge_body,
      out_type=[jax.ShapeDtypeStruct((E,), jnp.float32),
                jax.ShapeDtypeStruct((E,), jnp.float32)],
      mesh=plsc.VectorSubcoreMesh(core_axis_name="c", subcore_axis_name="s", num_cores=NC, num_subcores=NS),
      compiler_params=pltpu.CompilerParams(needs_layout_passes=False),
      scratch_types=(
          [pltpu.VMEM((EB,), jnp.int32)] * 4
          + [pltpu.VMEM((EB, D), jnp.float32)] * 6
          + [pltpu.VMEM((2, D), jnp.float32)]
          + [pltpu.VMEM((2, L), jnp.float32)]
          + [pltpu.VMEM((EB * L,), jnp.float32)] * 2
          + [pltpu.VMEM((EB,), jnp.float32)] * 4
          + [pltpu.SemaphoreType.DMA] * 6
      ),
  )(p, q, r, src, dst, w2, b2v)


_EBLK = 2000


def _tc_edge_lin1_body(ea_ref, w_ref, b_ref, o1_ref):
  o1_ref[...] = jnp.dot(ea_ref[...], w_ref[...],
                        preferred_element_type=jnp.float32) + b_ref[...]


def _tc_edge_lin2_body(ea_ref, w_ref, b_ref, o2_ref, o3_ref):
  acc = jnp.dot(ea_ref[...], w_ref[...],
                preferred_element_type=jnp.float32) + b_ref[...]
  o2_ref[...] = acc[:, :D]
  o3_ref[...] = acc[:, D:]


_e_out = jax.ShapeDtypeStruct((E, D), jnp.float32)


@jax.jit
def _tc_edge_lin1(ea_pad, w, b):
  return pl.pallas_call(
      _tc_edge_lin1_body,
      grid=(E // _EBLK,),
      in_specs=[
          pl.BlockSpec((_EBLK, 8), lambda i: (i, 0)),
          pl.BlockSpec((8, D), lambda i: (0, 0)),
          pl.BlockSpec((1, D), lambda i: (0, 0)),
      ],
      out_specs=pl.BlockSpec((_EBLK, D), lambda i: (i, 0)),
      out_shape=_e_out,
  )(ea_pad, w, b)


@jax.jit
def _tc_edge_lin2(ea_pad, w, b):
  return pl.pallas_call(
      _tc_edge_lin2_body,
      grid=(E // _EBLK,),
      in_specs=[
          pl.BlockSpec((_EBLK, 8), lambda i: (i, 0)),
          pl.BlockSpec((8, 2 * D), lambda i: (0, 0)),
          pl.BlockSpec((1, 2 * D), lambda i: (0, 0)),
      ],
      out_specs=[
          pl.BlockSpec((_EBLK, D), lambda i: (i, 0)),
          pl.BlockSpec((_EBLK, D), lambda i: (i, 0)),
      ],
      out_shape=[_e_out, _e_out],
  )(ea_pad, w, b)


_NBLK_TC = 1000


def _tc_mlp_body(x_ref, p_ref, w1_ref, b1_ref, w2_ref, b2_ref, o_ref):
  h = x_ref[...] + p_ref[0] + p_ref[1]
  t = jax.nn.relu(jnp.dot(h, w1_ref[...],
                          preferred_element_type=jnp.float32) + b1_ref[...])
  o_ref[...] = jax.nn.relu(jnp.dot(t, w2_ref[...],
                                   preferred_element_type=jnp.float32)
                           + b2_ref[...])


def _tc_mlp2_body(x_ref, p_ref, w1_ref, b1_ref, w2_ref, b2_ref,
                  ws_ref, wd_ref, o_ref, po_ref, qo_ref):
  h = x_ref[...] + p_ref[0] + p_ref[1]
  t = jax.nn.relu(jnp.dot(h, w1_ref[...],
                          preferred_element_type=jnp.float32) + b1_ref[...])
  o = jax.nn.relu(jnp.dot(t, w2_ref[...],
                          preferred_element_type=jnp.float32) + b2_ref[...])
  o_ref[...] = o
  po_ref[...] = jnp.dot(o, ws_ref[...], preferred_element_type=jnp.float32)
  qo_ref[...] = jnp.dot(o, wd_ref[...], preferred_element_type=jnp.float32)


_mat_spec = pl.BlockSpec((D, D), lambda i: (0, 0))
_bias_spec = pl.BlockSpec((1, D), lambda i: (0, 0))
_row_spec = pl.BlockSpec((_NBLK_TC, D), lambda i: (i, 0))
_part_spec = pl.BlockSpec((NC, _NBLK_TC, D), lambda i: (0, i, 0))
_n_out = jax.ShapeDtypeStruct((N, D), jnp.float32)


@jax.jit
def _tc_mlp(x, part, w1t, b1, w2t, b2):
  return pl.pallas_call(
      _tc_mlp_body,
      grid=(N // _NBLK_TC,),
      in_specs=[_row_spec, _part_spec, _mat_spec, _bias_spec, _mat_spec,
                _bias_spec],
      out_specs=_row_spec,
      out_shape=_n_out,
  )(x, part, w1t, b1, w2t, b2)


@jax.jit
def _tc_mlp2(x, part, w1t, b1, w2t, b2, wst, wdt):
  return pl.pallas_call(
      _tc_mlp2_body,
      grid=(N // _NBLK_TC,),
      in_specs=[_row_spec, _part_spec, _mat_spec, _bias_spec, _mat_spec,
                _bias_spec, _mat_spec, _mat_spec],
      out_specs=[_row_spec, _row_spec, _row_spec],
      out_shape=[_n_out, _n_out, _n_out],
  )(x, part, w1t, b1, w2t, b2, wst, wdt)


_SRCIDX = np.concatenate([
    np.stack([np.arange(16) + 32 * k2, np.arange(16) + 32 * k2 + 16],
             axis=1).reshape(-1)
    for k2 in range(D // 32)
])


def kernel(x, edge_index, edge_attr, e1_W, e1_b, n1_W1, n1_b1, n1_W2, n1_b2,
           e2_W, e2_b, n2_W1, n2_b1, n2_W2, n2_b2, m_W1, m_b1, m_W2, m_b2):
  src = edge_index[0]
  dst = edge_index[1]

  ea_pad = jnp.pad(edge_attr, ((0, 0), (0, 8 - ED)))
  we_t = m_W1[:, 2 * D:].T
  w1p = jnp.pad(e1_W.T, ((0, 1), (0, 0)))
  w23 = jnp.pad(jnp.concatenate([e2_W.T, we_t], axis=1), ((0, 1), (0, 0)))
  b23 = jnp.concatenate([e2_b, m_b1])[None, :]

  ep1 = _tc_edge_lin1(ea_pad, w1p, e1_b[None, :])

  part1 = _sc_agg(x, ep1, src, dst)
  ep2, r = _tc_edge_lin2(ea_pad, w23, b23)
  h1 = _tc_mlp(x, part1, n1_W1.T, n1_b1[None, :], n1_W2.T, n1_b2[None, :])

  part2 = _sc_agg(h1, ep2, src, dst)
  h2, p, q = _tc_mlp2(h1, part2, n2_W1.T, n2_b1[None, :], n2_W2.T,
                      n2_b2[None, :], m_W1[:, :D].T, m_W1[:, D:2 * D].T)

  b2v = jnp.broadcast_to(m_b2[:, None], (2, L))
  o0, o1 = _sc_edge(p, q, r, src, dst, m_W2, b2v)
  return jnp.stack([o0, o1], axis=1)


# speedup vs baseline: 1.1308x; 1.0528x over previous
import functools

import jax
import jax.numpy as jnp
import numpy as np
from jax import lax
from jax.experimental import pallas as pl
from jax.experimental.pallas import tpu as pltpu
from jax.experimental.pallas import tpu_sc as plsc

N = 10000
E = 320000
D = 128
ED = 7

NC = 2
NS = 16
L = 16
NW = NC * NS

EB = 80
EPT = E // NW
NB = EPT // EB
NPAD = 10240
ZROWS = NPAD // NS
OROWS = 624
NSEG = D // L


def _worker_id():
  return lax.axis_index("s") * NC + lax.axis_index("c")


def _sc_agg_body(h_hbm, ep_hbm, src_hbm, dst_hbm, out_hbm,
                 src0, src1, dst0, dst1, dsc0, dsc1, ep0, ep1, rows0, rows1,
                 acc_sh, si0, si1, sg0, sg1, so0, so1):
  c = lax.axis_index("c")
  s = lax.axis_index("s")
  wid = _worker_id()
  srcs, dsts, eps, rows = (src0, src1), (dst0, dst1), (ep0, ep1), (rows0, rows1)
  dscs = (dsc0, dsc1)
  sem_idx, sem_in, sem_out = (si0, si1), (sg0, sg1), (so0, so1)

  def off(g):
    return wid * EPT + g * EB

  def issue_idx(g, p):
    pltpu.async_copy(src_hbm.at[pl.ds(off(g), EB)], srcs[p], sem_idx[p])
    pltpu.async_copy(dst_hbm.at[pl.ds(off(g), EB)], dsts[p], sem_idx[p])

  def drain_idx(p):
    pltpu.make_async_copy(src_hbm.at[pl.ds(0, EB)], srcs[p], sem_idx[p]).wait()
    pltpu.make_async_copy(dst_hbm.at[pl.ds(0, EB)], dsts[p], sem_idx[p]).wait()

  def issue_in(g, p):
    pltpu.async_copy(h_hbm.at[srcs[p]], rows[p], sem_in[p])
    pltpu.async_copy(ep_hbm.at[pl.ds(off(g), EB)], eps[p], sem_in[p])

  def drain_in(p):
    pltpu.make_async_copy(h_hbm.at[srcs[p]], rows[p], sem_in[p]).wait()
    pltpu.make_async_copy(ep_hbm.at[pl.ds(0, EB)], eps[p], sem_in[p]).wait()

  def drain_out(p):
    pltpu.make_async_copy(rows[p], acc_sh.at[dscs[p]], sem_out[p]).wait()

  def copy_dst(p):
    for k in range(EB // L):
      dscs[p][pl.ds(k * L, L)] = dsts[p][pl.ds(k * L, L)]

  def compute(p):
    def edge(i, _):
      for k in range(NSEG):
        sl = pl.ds(k * L, L)
        rows[p][i, sl] = jnp.maximum(rows[p][i, sl] + eps[p][i, sl], 0.0)
      return 0
    lax.fori_loop(0, EB, edge, 0, unroll=False)

  def zrow(i, _):
    for k in range(NSEG):
      rows0[i, pl.ds(k * L, L)] = jnp.zeros((L,), jnp.float32)
    return 0
  lax.fori_loop(0, EB, zrow, 0, unroll=False)

  def zchunk(j, _):
    pltpu.sync_copy(rows0, acc_sh.at[pl.ds(s * ZROWS + j * EB, EB)])
    return 0
  lax.fori_loop(0, ZROWS // EB, zchunk, 0, unroll=False)
  plsc.subcore_barrier()

  issue_idx(0, 0)
  issue_idx(1, 1)
  drain_idx(0)
  issue_in(0, 0)

  def pair(gi, _):
    for p in range(2):
      g = gi * 2 + p
      q = 1 - p
      drain_in(p)
      copy_dst(p)

      @pl.when(g > 0)
      def _():
        drain_out(q)

      @pl.when(g < NB - 1)
      def _():
        drain_idx(q)
        issue_in(g + 1, q)

      @pl.when(g < NB - 2)
      def _():
        issue_idx(g + 2, p)

      compute(p)
      pltpu.async_copy(rows[p], acc_sh.at[dscs[p]], sem_out[p], add=True)
    return 0

  lax.fori_loop(0, NB // 2, pair, 0, unroll=False)

  drain_in(0)
  copy_dst(0)
  drain_out(1)
  compute(0)
  pltpu.sync_copy(rows0, acc_sh.at[dsc0], add=True)

  plsc.subcore_barrier()
  pltpu.sync_copy(acc_sh.at[pl.ds(s * OROWS, OROWS)],
                  out_hbm.at[c, pl.ds(s * OROWS, OROWS)])

  @pl.when(s == NS - 1)
  def _():
    pltpu.sync_copy(acc_sh.at[pl.ds(NS * OROWS, N - NS * OROWS)],
                    out_hbm.at[c, pl.ds(NS * OROWS, N - NS * OROWS)])


@jax.jit
def _sc_agg(h, ep, src, dst):
  return pl.kernel(
      _sc_agg_body,
      out_type=jax.ShapeDtypeStruct((NC, N, D), jnp.float32),
      mesh=plsc.VectorSubcoreMesh(core_axis_name="c", subcore_axis_name="s", num_cores=NC, num_subcores=NS),
      compiler_params=pltpu.CompilerParams(needs_layout_passes=False),
      scratch_types=(
          [pltpu.VMEM((EB,), jnp.int32)] * 6
          + [pltpu.VMEM((EB, D), jnp.float32)] * 4
          + [pltpu.VMEM_SHARED((NPAD, D), jnp.float32)]
          + [pltpu.SemaphoreType.DMA] * 6
      ),
  )(h, ep, src, dst)


def _sc_edge_body(p_hbm, q_hbm, r_hbm, src_hbm, dst_hbm, w2_hbm, b2_hbm,
                  o0_hbm, o1_hbm,
                  src0, src1, dst0, dst1, pv0, pv1, qv0, qv1, rv0, rv1,
                  w2_v, b2_v, m0_v, m1_v, o00, o01, o10, o11,
                  si0, si1, sg0, sg1, so0, so1):
  wid = _worker_id()
  srcs, dsts = (src0, src1), (dst0, dst1)
  pvs, qvs, rvs = (pv0, pv1), (qv0, qv1), (rv0, rv1)
  o0s, o1s = (o00, o01), (o10, o11)
  sem_idx, sem_in, sem_out = (si0, si1), (sg0, sg1), (so0, so1)
  pltpu.sync_copy(w2_hbm, w2_v)
  pltpu.sync_copy(b2_hbm, b2_v)

  def off(g):
    return wid * EPT + g * EB

  def issue_idx(g, p):
    pltpu.async_copy(src_hbm.at[pl.ds(off(g), EB)], srcs[p], sem_idx[p])
    pltpu.async_copy(dst_hbm.at[pl.ds(off(g), EB)], dsts[p], sem_idx[p])

  def drain_idx(p):
    pltpu.make_async_copy(src_hbm.at[pl.ds(0, EB)], srcs[p], sem_idx[p]).wait()
    pltpu.make_async_copy(dst_hbm.at[pl.ds(0, EB)], dsts[p], sem_idx[p]).wait()

  def issue_in(g, p):
    pltpu.async_copy(p_hbm.at[srcs[p]], pvs[p], sem_in[p])
    pltpu.async_copy(q_hbm.at[dsts[p]], qvs[p], sem_in[p])
    pltpu.async_copy(r_hbm.at[pl.ds(off(g), EB)], rvs[p], sem_in[p])

  def drain_in(p):
    pltpu.make_async_copy(p_hbm.at[srcs[p]], pvs[p], sem_in[p]).wait()
    pltpu.make_async_copy(q_hbm.at[dsts[p]], qvs[p], sem_in[p]).wait()
    pltpu.make_async_copy(r_hbm.at[pl.ds(0, EB)], rvs[p], sem_in[p]).wait()

  def drain_out(p):
    pltpu.make_async_copy(o0s[p], o0_hbm.at[pl.ds(0, EB)], sem_out[p]).wait()
    pltpu.make_async_copy(o1s[p], o1_hbm.at[pl.ds(0, EB)], sem_out[p]).wait()

  w2r = [w2_v[j, pl.ds(k * L, L)] for j in range(2) for k in range(NSEG)]
  b2r = (b2_v[0, :], b2_v[1, :])

  def compute(p):
    def edge(i, _):
      acc0 = jnp.zeros((L,), jnp.float32)
      acc1 = jnp.zeros((L,), jnp.float32)
      for k in range(NSEG):
        sl = pl.ds(k * L, L)
        z = jnp.maximum(pvs[p][i, sl] + qvs[p][i, sl] + rvs[p][i, sl], 0.0)
        acc0 = acc0 + z * w2r[k]
        acc1 = acc1 + z * w2r[NSEG + k]
      m0_v[pl.ds(i * L, L)] = acc0
      m1_v[pl.ds(i * L, L)] = acc1
      return 0
    lax.fori_loop(0, EB, edge, 0, unroll=False)

    def grp(jg, _):
      rows = (jnp.arange(L, dtype=jnp.int32) + jg * L) * L
      v0 = b2r[0]
      v1 = b2r[1]
      for k in range(L):
        idx = rows + k
        v0 = v0 + plsc.load_gather(m0_v, [idx])
        v1 = v1 + plsc.load_gather(m1_v, [idx])
      o0s[p][pl.ds(jg * L, L)] = v0
      o1s[p][pl.ds(jg * L, L)] = v1
      return 0
    lax.fori_loop(0, EB // L, grp, 0, unroll=False)

  issue_idx(0, 0)
  issue_idx(1, 1)
  drain_idx(0)
  issue_in(0, 0)

  def pair(gi, _):
    for p in range(2):
      g = gi * 2 + p
      q = 1 - p
      drain_in(p)

      @pl.when(g < NB - 1)
      def _():
        drain_idx(q)
        issue_in(g + 1, q)

      @pl.when(g < NB - 2)
      def _():
        issue_idx(g + 2, p)

      @pl.when(g > 1)
      def _():
        drain_out(p)

      compute(p)
      pltpu.async_copy(o0s[p], o0_hbm.at[pl.ds(off(g), EB)], sem_out[p])
      pltpu.async_copy(o1s[p], o1_hbm.at[pl.ds(off(g), EB)], sem_out[p])
    return 0

  lax.fori_loop(0, NB // 2, pair, 0, unroll=False)

  g = NB - 1
  drain_in(0)
  drain_out(0)
  compute(0)
  pltpu.sync_copy(o00, o0_hbm.at[pl.ds(off(g), EB)])
  pltpu.sync_copy(o10, o1_hbm.at[pl.ds(off(g), EB)])
  drain_out(1)


@jax.jit
def _sc_edge(p, q, r, src, dst, w2, b2v):
  return pl.kernel(
      _sc_edge_body,
      out_type=[jax.ShapeDtypeStruct((E,), jnp.float32),
                jax.ShapeDtypeStruct((E,), jnp.float32)],
      mesh=plsc.VectorSubcoreMesh(core_axis_name="c", subcore_axis_name="s", num_cores=NC, num_subcores=NS),
      compiler_params=pltpu.CompilerParams(needs_layout_passes=False),
      scratch_types=(
          [pltpu.VMEM((EB,), jnp.int32)] * 4
          + [pltpu.VMEM((EB, D), jnp.float32)] * 6
          + [pltpu.VMEM((2, D), jnp.float32)]
          + [pltpu.VMEM((2, L), jnp.float32)]
          + [pltpu.VMEM((EB * L,), jnp.float32)] * 2
          + [pltpu.VMEM((EB,), jnp.float32)] * 4
          + [pltpu.SemaphoreType.DMA] * 6
      ),
  )(p, q, r, src, dst, w2, b2v)


_EBLK = 2000


def _tc_edge_lin1_body(ea_ref, w_ref, b_ref, o1_ref):
  o1_ref[...] = jnp.dot(ea_ref[...], w_ref[...],
                        preferred_element_type=jnp.float32) + b_ref[...]


def _tc_edge_lin2_body(ea_ref, w_ref, b_ref, o2_ref, o3_ref):
  acc = jnp.dot(ea_ref[...], w_ref[...],
                preferred_element_type=jnp.float32) + b_ref[...]
  o2_ref[...] = acc[:, :D]
  o3_ref[...] = acc[:, D:]


_e_out = jax.ShapeDtypeStruct((E, D), jnp.float32)


@jax.jit
def _tc_edge_lin1(ea_pad, w, b):
  return pl.pallas_call(
      _tc_edge_lin1_body,
      grid=(E // _EBLK,),
      in_specs=[
          pl.BlockSpec((_EBLK, 8), lambda i: (i, 0)),
          pl.BlockSpec((8, D), lambda i: (0, 0)),
          pl.BlockSpec((1, D), lambda i: (0, 0)),
      ],
      out_specs=pl.BlockSpec((_EBLK, D), lambda i: (i, 0)),
      out_shape=_e_out,
  )(ea_pad, w, b)


@jax.jit
def _tc_edge_lin2(ea_pad, w, b):
  return pl.pallas_call(
      _tc_edge_lin2_body,
      grid=(E // _EBLK,),
      in_specs=[
          pl.BlockSpec((_EBLK, 8), lambda i: (i, 0)),
          pl.BlockSpec((8, 2 * D), lambda i: (0, 0)),
          pl.BlockSpec((1, 2 * D), lambda i: (0, 0)),
      ],
      out_specs=[
          pl.BlockSpec((_EBLK, D), lambda i: (i, 0)),
          pl.BlockSpec((_EBLK, D), lambda i: (i, 0)),
      ],
      out_shape=[_e_out, _e_out],
  )(ea_pad, w, b)


_NBLK_TC = 1000


def _tc_mlp_body(x_ref, p_ref, w1_ref, b1_ref, w2_ref, b2_ref, o_ref):
  h = x_ref[...] + p_ref[0] + p_ref[1]
  t = jax.nn.relu(jnp.dot(h, w1_ref[...],
                          preferred_element_type=jnp.float32) + b1_ref[...])
  o_ref[...] = jax.nn.relu(jnp.dot(t, w2_ref[...],
                                   preferred_element_type=jnp.float32)
                           + b2_ref[...])


def _tc_mlp2_body(x_ref, p_ref, w1_ref, b1_ref, w2_ref, b2_ref,
                  ws_ref, wd_ref, o_ref, po_ref, qo_ref):
  h = x_ref[...] + p_ref[0] + p_ref[1]
  t = jax.nn.relu(jnp.dot(h, w1_ref[...],
                          preferred_element_type=jnp.float32) + b1_ref[...])
  o = jax.nn.relu(jnp.dot(t, w2_ref[...],
                          preferred_element_type=jnp.float32) + b2_ref[...])
  o_ref[...] = o
  po_ref[...] = jnp.dot(o, ws_ref[...], preferred_element_type=jnp.float32)
  qo_ref[...] = jnp.dot(o, wd_ref[...], preferred_element_type=jnp.float32)


_mat_spec = pl.BlockSpec((D, D), lambda i: (0, 0))
_bias_spec = pl.BlockSpec((1, D), lambda i: (0, 0))
_row_spec = pl.BlockSpec((_NBLK_TC, D), lambda i: (i, 0))
_part_spec = pl.BlockSpec((NC, _NBLK_TC, D), lambda i: (0, i, 0))
_n_out = jax.ShapeDtypeStruct((N, D), jnp.float32)


@jax.jit
def _tc_mlp(x, part, w1t, b1, w2t, b2):
  return pl.pallas_call(
      _tc_mlp_body,
      grid=(N // _NBLK_TC,),
      in_specs=[_row_spec, _part_spec, _mat_spec, _bias_spec, _mat_spec,
                _bias_spec],
      out_specs=_row_spec,
      out_shape=_n_out,
  )(x, part, w1t, b1, w2t, b2)


@jax.jit
def _tc_mlp2(x, part, w1t, b1, w2t, b2, wst, wdt):
  return pl.pallas_call(
      _tc_mlp2_body,
      grid=(N // _NBLK_TC,),
      in_specs=[_row_spec, _part_spec, _mat_spec, _bias_spec, _mat_spec,
                _bias_spec, _mat_spec, _mat_spec],
      out_specs=[_row_spec, _row_spec, _row_spec],
      out_shape=[_n_out, _n_out, _n_out],
  )(x, part, w1t, b1, w2t, b2, wst, wdt)


_SRCIDX = np.concatenate([
    np.stack([np.arange(16) + 32 * k2, np.arange(16) + 32 * k2 + 16],
             axis=1).reshape(-1)
    for k2 in range(D // 32)
])


def kernel(x, edge_index, edge_attr, e1_W, e1_b, n1_W1, n1_b1, n1_W2, n1_b2,
           e2_W, e2_b, n2_W1, n2_b1, n2_W2, n2_b2, m_W1, m_b1, m_W2, m_b2):
  src = edge_index[0]
  dst = edge_index[1]

  ea_pad = jnp.pad(edge_attr, ((0, 0), (0, 8 - ED)))
  we_t = m_W1[:, 2 * D:].T
  w1p = jnp.pad(e1_W.T, ((0, 1), (0, 0)))
  w23 = jnp.pad(jnp.concatenate([e2_W.T, we_t], axis=1), ((0, 1), (0, 0)))
  b23 = jnp.concatenate([e2_b, m_b1])[None, :]

  ep1 = _tc_edge_lin1(ea_pad, w1p, e1_b[None, :])

  part1 = _sc_agg(x, ep1, src, dst)
  ep2, r = _tc_edge_lin2(ea_pad, w23, b23)
  h1 = _tc_mlp(x, part1, n1_W1.T, n1_b1[None, :], n1_W2.T, n1_b2[None, :])

  part2 = _sc_agg(h1, ep2, src, dst)
  h2, p, q = _tc_mlp2(h1, part2, n2_W1.T, n2_b1[None, :], n2_W2.T,
                      n2_b2[None, :], m_W1[:, :D].T, m_W1[:, D:2 * D].T)

  b2v = jnp.broadcast_to(m_b2[:, None], (2, L))
  o0, o1 = _sc_edge(p, q, r, src, dst, m_W2, b2v)
  return jnp.stack([o0, o1], axis=1)


# speedup vs baseline: 1.1656x; 1.0308x over previous
import functools

import jax
import jax.numpy as jnp
import numpy as np
from jax import lax
from jax.experimental import pallas as pl
from jax.experimental.pallas import tpu as pltpu
from jax.experimental.pallas import tpu_sc as plsc

N = 10000
E = 320000
D = 128
ED = 7

NC = 2
NS = 16
L = 16
NW = NC * NS

EB = 80
EPT = E // NW
NB = EPT // EB
NPAD = 10240
ZROWS = NPAD // NS
OROWS = 624
NSEG = D // L


def _worker_id():
  return lax.axis_index("s") * NC + lax.axis_index("c")


def _sc_agg_body(h_hbm, ep_hbm, src_hbm, dst_hbm, out_hbm,
                 src0, src1, dst0, dst1, dsc0, dsc1, ep0, ep1, rows0, rows1,
                 acc_sh, si0, si1, sg0, sg1, so0, so1):
  c = lax.axis_index("c")
  s = lax.axis_index("s")
  wid = _worker_id()
  srcs, dsts, eps, rows = (src0, src1), (dst0, dst1), (ep0, ep1), (rows0, rows1)
  dscs = (dsc0, dsc1)
  sem_idx, sem_in, sem_out = (si0, si1), (sg0, sg1), (so0, so1)

  def off(g):
    return wid * EPT + g * EB

  def issue_idx(g, p):
    pltpu.async_copy(src_hbm.at[pl.ds(off(g), EB)], srcs[p], sem_idx[p])
    pltpu.async_copy(dst_hbm.at[pl.ds(off(g), EB)], dsts[p], sem_idx[p])

  def drain_idx(p):
    pltpu.make_async_copy(src_hbm.at[pl.ds(0, EB)], srcs[p], sem_idx[p]).wait()
    pltpu.make_async_copy(dst_hbm.at[pl.ds(0, EB)], dsts[p], sem_idx[p]).wait()

  def issue_in(g, p):
    pltpu.async_copy(h_hbm.at[srcs[p]], rows[p], sem_in[p])
    pltpu.async_copy(ep_hbm.at[pl.ds(off(g), EB)], eps[p], sem_in[p])

  def drain_in(p):
    pltpu.make_async_copy(h_hbm.at[srcs[p]], rows[p], sem_in[p]).wait()
    pltpu.make_async_copy(ep_hbm.at[pl.ds(0, EB)], eps[p], sem_in[p]).wait()

  def drain_out(p):
    pltpu.make_async_copy(rows[p], acc_sh.at[dscs[p]], sem_out[p]).wait()

  def copy_dst(p):
    for k in range(EB // L):
      dscs[p][pl.ds(k * L, L)] = dsts[p][pl.ds(k * L, L)]

  def compute(p):
    @plsc.parallel_loop(0, EB, unroll=2)
    def edge(i):
      for k in range(NSEG):
        sl = pl.ds(k * L, L)
        rows[p][i, sl] = jnp.maximum(rows[p][i, sl] + eps[p][i, sl], 0.0)

  def zrow(i, _):
    for k in range(NSEG):
      rows0[i, pl.ds(k * L, L)] = jnp.zeros((L,), jnp.float32)
    return 0
  lax.fori_loop(0, EB, zrow, 0, unroll=False)

  def zchunk(j, _):
    pltpu.sync_copy(rows0, acc_sh.at[pl.ds(s * ZROWS + j * EB, EB)])
    return 0
  lax.fori_loop(0, ZROWS // EB, zchunk, 0, unroll=False)
  plsc.subcore_barrier()

  issue_idx(0, 0)
  issue_idx(1, 1)
  drain_idx(0)
  issue_in(0, 0)

  def pair(gi, _):
    for p in range(2):
      g = gi * 2 + p
      q = 1 - p
      drain_in(p)
      copy_dst(p)

      @pl.when(g > 0)
      def _():
        drain_out(q)

      @pl.when(g < NB - 1)
      def _():
        drain_idx(q)
        issue_in(g + 1, q)

      @pl.when(g < NB - 2)
      def _():
        issue_idx(g + 2, p)

      compute(p)
      pltpu.async_copy(rows[p], acc_sh.at[dscs[p]], sem_out[p], add=True)
    return 0

  lax.fori_loop(0, NB // 2, pair, 0, unroll=False)

  drain_in(0)
  copy_dst(0)
  drain_out(1)
  compute(0)
  pltpu.sync_copy(rows0, acc_sh.at[dsc0], add=True)

  plsc.subcore_barrier()
  pltpu.sync_copy(acc_sh.at[pl.ds(s * OROWS, OROWS)],
                  out_hbm.at[c, pl.ds(s * OROWS, OROWS)])

  @pl.when(s == NS - 1)
  def _():
    pltpu.sync_copy(acc_sh.at[pl.ds(NS * OROWS, N - NS * OROWS)],
                    out_hbm.at[c, pl.ds(NS * OROWS, N - NS * OROWS)])


@jax.jit
def _sc_agg(h, ep, src, dst):
  return pl.kernel(
      _sc_agg_body,
      out_type=jax.ShapeDtypeStruct((NC, N, D), jnp.float32),
      mesh=plsc.VectorSubcoreMesh(core_axis_name="c", subcore_axis_name="s", num_cores=NC, num_subcores=NS),
      compiler_params=pltpu.CompilerParams(needs_layout_passes=False),
      scratch_types=(
          [pltpu.VMEM((EB,), jnp.int32)] * 6
          + [pltpu.VMEM((EB, D), jnp.float32)] * 4
          + [pltpu.VMEM_SHARED((NPAD, D), jnp.float32)]
          + [pltpu.SemaphoreType.DMA] * 6
      ),
  )(h, ep, src, dst)


def _sc_edge_body(p_hbm, q_hbm, r_hbm, src_hbm, dst_hbm, w2_hbm, b2_hbm,
                  o0_hbm, o1_hbm,
                  src0, src1, dst0, dst1, pv0, pv1, qv0, qv1, rv0, rv1,
                  w2_v, b2_v, m0_v, m1_v, o00, o01, o10, o11,
                  si0, si1, sg0, sg1, so0, so1):
  wid = _worker_id()
  srcs, dsts = (src0, src1), (dst0, dst1)
  pvs, qvs, rvs = (pv0, pv1), (qv0, qv1), (rv0, rv1)
  o0s, o1s = (o00, o01), (o10, o11)
  sem_idx, sem_in, sem_out = (si0, si1), (sg0, sg1), (so0, so1)
  pltpu.sync_copy(w2_hbm, w2_v)
  pltpu.sync_copy(b2_hbm, b2_v)

  def off(g):
    return wid * EPT + g * EB

  def issue_idx(g, p):
    pltpu.async_copy(src_hbm.at[pl.ds(off(g), EB)], srcs[p], sem_idx[p])
    pltpu.async_copy(dst_hbm.at[pl.ds(off(g), EB)], dsts[p], sem_idx[p])

  def drain_idx(p):
    pltpu.make_async_copy(src_hbm.at[pl.ds(0, EB)], srcs[p], sem_idx[p]).wait()
    pltpu.make_async_copy(dst_hbm.at[pl.ds(0, EB)], dsts[p], sem_idx[p]).wait()

  def issue_in(g, p):
    pltpu.async_copy(p_hbm.at[srcs[p]], pvs[p], sem_in[p])
    pltpu.async_copy(q_hbm.at[dsts[p]], qvs[p], sem_in[p])
    pltpu.async_copy(r_hbm.at[pl.ds(off(g), EB)], rvs[p], sem_in[p])

  def drain_in(p):
    pltpu.make_async_copy(p_hbm.at[srcs[p]], pvs[p], sem_in[p]).wait()
    pltpu.make_async_copy(q_hbm.at[dsts[p]], qvs[p], sem_in[p]).wait()
    pltpu.make_async_copy(r_hbm.at[pl.ds(0, EB)], rvs[p], sem_in[p]).wait()

  def drain_out(p):
    pltpu.make_async_copy(o0s[p], o0_hbm.at[pl.ds(0, EB)], sem_out[p]).wait()
    pltpu.make_async_copy(o1s[p], o1_hbm.at[pl.ds(0, EB)], sem_out[p]).wait()

  w2r = [w2_v[j, pl.ds(k * L, L)] for j in range(2) for k in range(NSEG)]
  b2r = (b2_v[0, :], b2_v[1, :])

  def compute(p):
    @plsc.parallel_loop(0, EB, unroll=2)
    def edge(i):
      acc0 = jnp.zeros((L,), jnp.float32)
      acc1 = jnp.zeros((L,), jnp.float32)
      for k in range(NSEG):
        sl = pl.ds(k * L, L)
        z = jnp.maximum(pvs[p][i, sl] + qvs[p][i, sl] + rvs[p][i, sl], 0.0)
        acc0 = acc0 + z * w2r[k]
        acc1 = acc1 + z * w2r[NSEG + k]
      m0_v[pl.ds(i * L, L)] = acc0
      m1_v[pl.ds(i * L, L)] = acc1

    def grp(jg, _):
      rows = (jnp.arange(L, dtype=jnp.int32) + jg * L) * L
      v0 = b2r[0]
      v1 = b2r[1]
      for k in range(L):
        idx = rows + k
        v0 = v0 + plsc.load_gather(m0_v, [idx])
        v1 = v1 + plsc.load_gather(m1_v, [idx])
      o0s[p][pl.ds(jg * L, L)] = v0
      o1s[p][pl.ds(jg * L, L)] = v1
      return 0
    lax.fori_loop(0, EB // L, grp, 0, unroll=False)

  issue_idx(0, 0)
  issue_idx(1, 1)
  drain_idx(0)
  issue_in(0, 0)

  def pair(gi, _):
    for p in range(2):
      g = gi * 2 + p
      q = 1 - p
      drain_in(p)

      @pl.when(g < NB - 1)
      def _():
        drain_idx(q)
        issue_in(g + 1, q)

      @pl.when(g < NB - 2)
      def _():
        issue_idx(g + 2, p)

      @pl.when(g > 1)
      def _():
        drain_out(p)

      compute(p)
      pltpu.async_copy(o0s[p], o0_hbm.at[pl.ds(off(g), EB)], sem_out[p])
      pltpu.async_copy(o1s[p], o1_hbm.at[pl.ds(off(g), EB)], sem_out[p])
    return 0

  lax.fori_loop(0, NB // 2, pair, 0, unroll=False)

  g = NB - 1
  drain_in(0)
  drain_out(0)
  compute(0)
  pltpu.sync_copy(o00, o0_hbm.at[pl.ds(off(g), EB)])
  pltpu.sync_copy(o10, o1_hbm.at[pl.ds(off(g), EB)])
  drain_out(1)


@jax.jit
def _sc_edge(p, q, r, src, dst, w2, b2v):
  return pl.kernel(
      _sc_edge_body,
      out_type=[jax.ShapeDtypeStruct((E,), jnp.float32),
                jax.ShapeDtypeStruct((E,), jnp.float32)],
      mesh=plsc.VectorSubcoreMesh(core_axis_name="c", subcore_axis_name="s", num_cores=NC, num_subcores=NS),
      compiler_params=pltpu.CompilerParams(needs_layout_passes=False),
      scratch_types=(
          [pltpu.VMEM((EB,), jnp.int32)] * 4
          + [pltpu.VMEM((EB, D), jnp.float32)] * 6
          + [pltpu.VMEM((2, D), jnp.float32)]
          + [pltpu.VMEM((2, L), jnp.float32)]
          + [pltpu.VMEM((EB * L,), jnp.float32)] * 2
          + [pltpu.VMEM((EB,), jnp.float32)] * 4
          + [pltpu.SemaphoreType.DMA] * 6
      ),
  )(p, q, r, src, dst, w2, b2v)


_EBLK = 2000


def _tc_edge_lin1_body(ea_ref, w_ref, b_ref, o1_ref):
  o1_ref[...] = jnp.dot(ea_ref[...], w_ref[...],
                        preferred_element_type=jnp.float32) + b_ref[...]


def _tc_edge_lin2_body(ea_ref, w_ref, b_ref, o2_ref, o3_ref):
  acc = jnp.dot(ea_ref[...], w_ref[...],
                preferred_element_type=jnp.float32) + b_ref[...]
  o2_ref[...] = acc[:, :D]
  o3_ref[...] = acc[:, D:]


_e_out = jax.ShapeDtypeStruct((E, D), jnp.float32)


@jax.jit
def _tc_edge_lin1(ea_pad, w, b):
  return pl.pallas_call(
      _tc_edge_lin1_body,
      grid=(E // _EBLK,),
      in_specs=[
          pl.BlockSpec((_EBLK, 8), lambda i: (i, 0)),
          pl.BlockSpec((8, D), lambda i: (0, 0)),
          pl.BlockSpec((1, D), lambda i: (0, 0)),
      ],
      out_specs=pl.BlockSpec((_EBLK, D), lambda i: (i, 0)),
      out_shape=_e_out,
  )(ea_pad, w, b)


@jax.jit
def _tc_edge_lin2(ea_pad, w, b):
  return pl.pallas_call(
      _tc_edge_lin2_body,
      grid=(E // _EBLK,),
      in_specs=[
          pl.BlockSpec((_EBLK, 8), lambda i: (i, 0)),
          pl.BlockSpec((8, 2 * D), lambda i: (0, 0)),
          pl.BlockSpec((1, 2 * D), lambda i: (0, 0)),
      ],
      out_specs=[
          pl.BlockSpec((_EBLK, D), lambda i: (i, 0)),
          pl.BlockSpec((_EBLK, D), lambda i: (i, 0)),
      ],
      out_shape=[_e_out, _e_out],
  )(ea_pad, w, b)


_NBLK_TC = 1000


def _tc_mlp_body(x_ref, p_ref, w1_ref, b1_ref, w2_ref, b2_ref, o_ref):
  h = x_ref[...] + p_ref[0] + p_ref[1]
  t = jax.nn.relu(jnp.dot(h, w1_ref[...],
                          preferred_element_type=jnp.float32) + b1_ref[...])
  o_ref[...] = jax.nn.relu(jnp.dot(t, w2_ref[...],
                                   preferred_element_type=jnp.float32)
                           + b2_ref[...])


def _tc_mlp2_body(x_ref, p_ref, w1_ref, b1_ref, w2_ref, b2_ref,
                  ws_ref, wd_ref, o_ref, po_ref, qo_ref):
  h = x_ref[...] + p_ref[0] + p_ref[1]
  t = jax.nn.relu(jnp.dot(h, w1_ref[...],
                          preferred_element_type=jnp.float32) + b1_ref[...])
  o = jax.nn.relu(jnp.dot(t, w2_ref[...],
                          preferred_element_type=jnp.float32) + b2_ref[...])
  o_ref[...] = o
  po_ref[...] = jnp.dot(o, ws_ref[...], preferred_element_type=jnp.float32)
  qo_ref[...] = jnp.dot(o, wd_ref[...], preferred_element_type=jnp.float32)


_mat_spec = pl.BlockSpec((D, D), lambda i: (0, 0))
_bias_spec = pl.BlockSpec((1, D), lambda i: (0, 0))
_row_spec = pl.BlockSpec((_NBLK_TC, D), lambda i: (i, 0))
_part_spec = pl.BlockSpec((NC, _NBLK_TC, D), lambda i: (0, i, 0))
_n_out = jax.ShapeDtypeStruct((N, D), jnp.float32)


@jax.jit
def _tc_mlp(x, part, w1t, b1, w2t, b2):
  return pl.pallas_call(
      _tc_mlp_body,
      grid=(N // _NBLK_TC,),
      in_specs=[_row_spec, _part_spec, _mat_spec, _bias_spec, _mat_spec,
                _bias_spec],
      out_specs=_row_spec,
      out_shape=_n_out,
  )(x, part, w1t, b1, w2t, b2)


@jax.jit
def _tc_mlp2(x, part, w1t, b1, w2t, b2, wst, wdt):
  return pl.pallas_call(
      _tc_mlp2_body,
      grid=(N // _NBLK_TC,),
      in_specs=[_row_spec, _part_spec, _mat_spec, _bias_spec, _mat_spec,
                _bias_spec, _mat_spec, _mat_spec],
      out_specs=[_row_spec, _row_spec, _row_spec],
      out_shape=[_n_out, _n_out, _n_out],
  )(x, part, w1t, b1, w2t, b2, wst, wdt)


_SRCIDX = np.concatenate([
    np.stack([np.arange(16) + 32 * k2, np.arange(16) + 32 * k2 + 16],
             axis=1).reshape(-1)
    for k2 in range(D // 32)
])


def kernel(x, edge_index, edge_attr, e1_W, e1_b, n1_W1, n1_b1, n1_W2, n1_b2,
           e2_W, e2_b, n2_W1, n2_b1, n2_W2, n2_b2, m_W1, m_b1, m_W2, m_b2):
  src = edge_index[0]
  dst = edge_index[1]

  ea_pad = jnp.pad(edge_attr, ((0, 0), (0, 8 - ED)))
  we_t = m_W1[:, 2 * D:].T
  w1p = jnp.pad(e1_W.T, ((0, 1), (0, 0)))
  w23 = jnp.pad(jnp.concatenate([e2_W.T, we_t], axis=1), ((0, 1), (0, 0)))
  b23 = jnp.concatenate([e2_b, m_b1])[None, :]

  ep1 = _tc_edge_lin1(ea_pad, w1p, e1_b[None, :])

  part1 = _sc_agg(x, ep1, src, dst)
  ep2, r = _tc_edge_lin2(ea_pad, w23, b23)
  h1 = _tc_mlp(x, part1, n1_W1.T, n1_b1[None, :], n1_W2.T, n1_b2[None, :])

  part2 = _sc_agg(h1, ep2, src, dst)
  h2, p, q = _tc_mlp2(h1, part2, n2_W1.T, n2_b1[None, :], n2_W2.T,
                      n2_b2[None, :], m_W1[:, :D].T, m_W1[:, D:2 * D].T)

  b2v = jnp.broadcast_to(m_b2[:, None], (2, L))
  o0, o1 = _sc_edge(p, q, r, src, dst, m_W2, b2v)
  return jnp.stack([o0, o1], axis=1)
